# baseline (speedup 1.0000x reference)
import functools

import jax
import jax.numpy as jnp
from jax import lax
from jax.experimental import pallas as pl
from jax.experimental.pallas import tpu as pltpu
from jax.experimental.pallas import tpu_sc as plsc

NC = 2
NS = 16
NW = NC * NS
LANES = 16


def _dense_body(attr_ref, w_ref, b_ref, o_ref):
    o_ref[...] = lax.dot_general(
        attr_ref[...], w_ref[...],
        dimension_numbers=(((1,), (1,)), ((), ())),
        preferred_element_type=jnp.float32,
    ) + b_ref[...]


def _make_sc_kernel(Bsz, F, V, D):
    RPW = Bsz // NW
    R = 64
    NCH = RPW // R
    CL = R * F
    SEG = 128
    NSEG = CL // SEG
    assert CL % SEG == 0 and CL % LANES == 0 and D % LANES == 0
    CV = D // LANES

    mesh = plsc.VectorSubcoreMesh(
        core_axis_name="c", subcore_axis_name="s",
        num_cores=NC, num_subcores=NS,
    )

    @functools.partial(
        pl.kernel,
        out_type=jax.ShapeDtypeStruct((Bsz, D), jnp.float32),
        mesh=mesh,
        compiler_params=pltpu.CompilerParams(use_tc_tiling_on_sc=False),
        scratch_types=[
            pltpu.VMEM((CL,), jnp.int32),
            pltpu.VMEM((CL,), jnp.int32),
            pltpu.VMEM((CL,), jnp.int32),
            pltpu.VMEM((CL, D), jnp.float32),
            pltpu.VMEM((R, D), jnp.float32),
            pltpu.VMEM((R, D), jnp.float32),
            pltpu.SemaphoreType.DMA,
        ],
    )
    def sc_kernel(labels_hbm, offs_hbm, table_hbm, base_hbm, out_hbm,
                  lab_v, idx_v, offs_v, rows_v, base_v, out_v, sem):
        cid = lax.axis_index("c")
        sid = lax.axis_index("s")
        wid = sid * NC + cid
        row0 = wid * RPW

        pltpu.sync_copy(offs_hbm, offs_v)

        for g in range(NCH):
            r0 = row0 + g * R

            pltpu.sync_copy(labels_hbm.at[pl.ds(r0 * F, CL)], lab_v)

            def idx_body(p, _):
                q = p * LANES
                idx_v[pl.ds(q, LANES)] = (
                    lab_v[pl.ds(q, LANES)] + offs_v[pl.ds(q, LANES)]
                )
                return 0
            lax.fori_loop(0, CL // LANES, idx_body, 0)

            pltpu.sync_copy(base_hbm.at[pl.ds(r0, R)], base_v)

            handles = [
                pltpu.async_copy(
                    table_hbm.at[idx_v.at[pl.ds(s * SEG, SEG)]],
                    rows_v.at[pl.ds(s * SEG, SEG)],
                    sem,
                )
                for s in range(NSEG)
            ]
            for h in handles:
                h.wait()

            def row_body(r, _):
                rb = r * F
                accs = [base_v[r, pl.ds(cc * LANES, LANES)]
                        for cc in range(CV)]
                for j in range(F):
                    for cc in range(CV):
                        accs[cc] = accs[cc] + rows_v[rb + j,
                                                     pl.ds(cc * LANES, LANES)]
                for cc in range(CV):
                    out_v[r, pl.ds(cc * LANES, LANES)] = accs[cc]
                return 0
            lax.fori_loop(0, R, row_body, 0)

            pltpu.sync_copy(out_v, out_hbm.at[pl.ds(r0, R)])

    return sc_kernel


@jax.jit
def kernel(labels, attr, tables, W, b):
    Bsz, F = labels.shape
    _, V, D = tables.shape

    base = pl.pallas_call(
        _dense_body,
        out_shape=jax.ShapeDtypeStruct((Bsz, D), jnp.float32),
    )(attr, W, b.reshape(1, D))

    labels_flat = labels.astype(jnp.int32).reshape(-1)
    table_flat = tables.reshape(F * V, D)
    R = 64
    offs = jnp.tile(jnp.arange(F, dtype=jnp.int32) * V, R)

    sc = _make_sc_kernel(Bsz, F, V, D)
    return sc(labels_flat, offs, table_flat, base)
